# Initial kernel scaffold; baseline (speedup 1.0000x reference)
import jax
import jax.numpy as jnp
from jax.experimental import pallas as pl


def kernel(words, emb_table, W, b):
    raise NotImplementedError("write your pallas kernel here")


# speedup vs baseline: 4.4846x; 4.4846x over previous
import functools

import jax
import jax.numpy as jnp
from jax import lax
from jax.experimental import pallas as pl
from jax.experimental.pallas import tpu as pltpu
from jax.experimental.pallas import tpu_sc as plsc

_VOCAB, _EMB, _ASP = 1000, 64, 9
_B, _L = 4096, 200
_N = _B * _L

_INFO = plsc.get_sparse_core_info()
_NC, _NS = _INFO.num_cores, _INFO.num_subcores
_NW = _NC * _NS
_LANES = 16
_TOK_PER_W = _N // _NW
_CHTOK = 2560
_NCHUNK = _TOK_PER_W // _CHTOK
_NGRP = _CHTOK // _LANES
_OUT_CH = _CHTOK * _ASP


def _proj_body(emb_ref, w_ref, b_ref, out_ref):
    out_ref[...] = (
        jnp.dot(emb_ref[...], w_ref[...], preferred_element_type=jnp.float32)
        + b_ref[...]
    )


def _gather_body(proj_hbm, idx_hbm, out_hbm,
                 proj_v, idx_a, idx_b, out_a, out_b, sem_a, sem_b):
    wid = lax.axis_index("s") * _NC + lax.axis_index("c")
    tok0 = wid * _TOK_PER_W
    lane9 = lax.iota(jnp.int32, _LANES) * _ASP

    pltpu.sync_copy(proj_hbm, proj_v)

    def compute_chunk(idx_v, out_v):
        def group(i, carry):
            tok = idx_v[pl.ds(i * _LANES, _LANES)]
            t9 = tok * _ASP
            obase = lane9 + i * (_LANES * _ASP)
            for a in range(_ASP):
                vals = plsc.load_gather(proj_v, [t9 + a])
                plsc.store_scatter(out_v, [obase + a], vals)
            return carry

        lax.fori_loop(0, _NGRP, group, 0)

    def outer(co, carry):
        for bsel in range(2):
            c = co * 2 + bsel
            idx_v = idx_a if bsel == 0 else idx_b
            out_v = out_a if bsel == 0 else out_b
            sem = sem_a if bsel == 0 else sem_b
            start = tok0 + c * _CHTOK
            pltpu.sync_copy(idx_hbm.at[pl.ds(start, _CHTOK)], idx_v)
            @pl.when(co > 0)
            def _():
                pltpu.make_async_copy(
                    out_hbm.at[pl.ds(0, _OUT_CH)], out_v, sem
                ).wait()

            compute_chunk(idx_v, out_v)
            pltpu.async_copy(
                out_v, out_hbm.at[pl.ds(start * _ASP, _OUT_CH)], sem
            )
        return carry

    lax.fori_loop(0, _NCHUNK // 2, outer, 0)

    pltpu.make_async_copy(out_hbm.at[pl.ds(0, _OUT_CH)], out_a, sem_a).wait()
    pltpu.make_async_copy(out_hbm.at[pl.ds(0, _OUT_CH)], out_b, sem_b).wait()


_gather = functools.partial(
    pl.kernel,
    out_type=jax.ShapeDtypeStruct((_N * _ASP,), jnp.float32),
    mesh=plsc.VectorSubcoreMesh(core_axis_name="c", subcore_axis_name="s"),
    compiler_params=pltpu.CompilerParams(needs_layout_passes=False),
    scratch_types=[
        pltpu.VMEM((_VOCAB * _ASP,), jnp.float32),
        pltpu.VMEM((_CHTOK,), jnp.int32),
        pltpu.VMEM((_CHTOK,), jnp.int32),
        pltpu.VMEM((_OUT_CH,), jnp.float32),
        pltpu.VMEM((_OUT_CH,), jnp.float32),
        pltpu.SemaphoreType.DMA,
        pltpu.SemaphoreType.DMA,
    ],
)(_gather_body)


def kernel(words, emb_table, W, b):
    proj = pl.pallas_call(
        _proj_body,
        out_shape=jax.ShapeDtypeStruct((_VOCAB, _ASP), jnp.float32),
    )(emb_table, W, b.reshape(1, _ASP))
    out = _gather(proj.reshape(_VOCAB * _ASP), words.reshape(_N))
    return out.reshape(_B, _L, _ASP)


# speedup vs baseline: 20.7426x; 4.6253x over previous
import functools

import jax
import jax.numpy as jnp
from jax import lax
from jax.experimental import pallas as pl
from jax.experimental.pallas import tpu as pltpu
from jax.experimental.pallas import tpu_sc as plsc

_VOCAB, _EMB, _ASP = 1000, 64, 9
_B, _L = 4096, 200

_INFO = plsc.get_sparse_core_info()
_NC, _NS = _INFO.num_cores, _INFO.num_subcores
_NW = _NC * _NS
_LANES = 16
_BPW = _B // _NW
_LCH = 8
_NCHUNK = _L // _LCH
_NBG = _BPW // _LANES


def _proj_body(emb_ref, w_ref, b_ref, out_ref):
    out_ref[...] = (
        jnp.dot(emb_ref[...], w_ref[...], preferred_element_type=jnp.float32)
        + b_ref[...]
    )


def _gather_body(proj_hbm, idx_hbm, out_hbm,
                 proj_v, idx_v, out_a, out_b, sem_a, sem_b):
    wid = lax.axis_index("s") * _NC + lax.axis_index("c")
    b0 = wid * _BPW
    iota200 = lax.iota(jnp.int32, _LANES) * _L

    pltpu.sync_copy(proj_hbm, proj_v)
    pltpu.sync_copy(idx_hbm.at[pl.ds(b0 * _L, _BPW * _L)], idx_v)

    def compute_chunk(lc, outv):
        l0 = lc * _LCH
        for l in range(_LCH):
            for bg in range(_NBG):
                tok = plsc.load_gather(
                    idx_v, [iota200 + (l0 + bg * _LANES * _L + l)]
                )
                t9 = tok * _ASP
                for a in range(_ASP):
                    vals = plsc.load_gather(proj_v, [t9 + a])
                    outv[a, l, pl.ds(bg * _LANES, _LANES)] = vals

    def store_chunk(lc, outv, sem):
        pltpu.async_copy(
            outv, out_hbm.at[:, pl.ds(lc * _LCH, _LCH), pl.ds(b0, _BPW)], sem
        )

    def drain(outv, sem):
        pltpu.make_async_copy(
            out_hbm.at[:, pl.ds(0, _LCH), pl.ds(0, _BPW)], outv, sem
        ).wait()

    def outer(p, carry):
        for par in range(2):
            lc = p * 2 + par
            outv = out_a if par == 0 else out_b
            sem = sem_a if par == 0 else sem_b

            @pl.when(p > 0)
            def _():
                drain(outv, sem)

            compute_chunk(lc, outv)
            store_chunk(lc, outv, sem)
        return carry

    lax.fori_loop(0, (_NCHUNK - 1) // 2, outer, 0)
    drain(out_a, sem_a)
    compute_chunk(jnp.int32(_NCHUNK - 1), out_a)
    store_chunk(jnp.int32(_NCHUNK - 1), out_a, sem_a)
    drain(out_a, sem_a)
    drain(out_b, sem_b)


_gather = functools.partial(
    pl.kernel,
    out_type=jax.ShapeDtypeStruct((_ASP, _L, _B), jnp.float32),
    mesh=plsc.VectorSubcoreMesh(core_axis_name="c", subcore_axis_name="s"),
    compiler_params=pltpu.CompilerParams(needs_layout_passes=False),
    scratch_types=[
        pltpu.VMEM((_VOCAB * _ASP,), jnp.float32),
        pltpu.VMEM((_BPW * _L,), jnp.int32),
        pltpu.VMEM((_ASP, _LCH, _BPW), jnp.float32),
        pltpu.VMEM((_ASP, _LCH, _BPW), jnp.float32),
        pltpu.SemaphoreType.DMA,
        pltpu.SemaphoreType.DMA,
    ],
)(_gather_body)


def kernel(words, emb_table, W, b):
    proj = pl.pallas_call(
        _proj_body,
        out_shape=jax.ShapeDtypeStruct((_VOCAB, _ASP), jnp.float32),
    )(emb_table, W, b.reshape(1, _ASP))
    out_t = _gather(proj.reshape(_VOCAB * _ASP), words.reshape(_B * _L))
    return jnp.transpose(out_t, (2, 1, 0))


# speedup vs baseline: 38.7837x; 1.8698x over previous
import functools

import jax
import jax.numpy as jnp
from jax import lax
from jax.experimental import pallas as pl
from jax.experimental.pallas import tpu as pltpu
from jax.experimental.pallas import tpu_sc as plsc

_VOCAB, _EMB, _ASP = 1000, 64, 9
_B, _L = 4096, 200

_INFO = plsc.get_sparse_core_info()
_NC, _NS = _INFO.num_cores, _INFO.num_subcores
_NW = _NC * _NS
_LANES = 16
_BPW = _B // _NW
_LCH = 8
_NCHUNK = _L // _LCH
_NBG = _BPW // _LANES


def _proj_body(emb_ref, w_ref, b_ref, out_ref):
    out_ref[...] = (
        jnp.dot(emb_ref[...], w_ref[...], preferred_element_type=jnp.float32)
        + b_ref[...]
    )


def _gather_body(proj_hbm, idx_hbm, out_hbm,
                 proj_v, idx_v, out_a, out_b, sem_a, sem_b):
    wid = lax.axis_index("s") * _NC + lax.axis_index("c")
    b0 = wid * _BPW
    iota200 = lax.iota(jnp.int32, _LANES) * _L

    pltpu.sync_copy(proj_hbm, proj_v)
    pltpu.sync_copy(idx_hbm.at[pl.ds(b0 * _L, _BPW * _L)], idx_v)

    def compute_chunk(lc, outv):
        l0 = lc * _LCH

        @plsc.parallel_loop(0, _NBG, unroll=4)
        def _(bg):
            base = l0 + bg * (_LANES * _L)
            for l in range(_LCH):
                tok = plsc.load_gather(idx_v, [iota200 + (base + l)])
                t9 = tok * _ASP
                for a in range(_ASP):
                    vals = plsc.load_gather(proj_v, [t9 + a])
                    outv[a, l, pl.ds(bg * _LANES, _LANES)] = vals

    def store_chunk(lc, outv, sem):
        pltpu.async_copy(
            outv, out_hbm.at[:, pl.ds(lc * _LCH, _LCH), pl.ds(b0, _BPW)], sem
        )

    def drain(outv, sem):
        pltpu.make_async_copy(
            out_hbm.at[:, pl.ds(0, _LCH), pl.ds(0, _BPW)], outv, sem
        ).wait()

    def outer(p, carry):
        for par in range(2):
            lc = p * 2 + par
            outv = out_a if par == 0 else out_b
            sem = sem_a if par == 0 else sem_b

            @pl.when(p > 0)
            def _():
                drain(outv, sem)

            compute_chunk(lc, outv)
            store_chunk(lc, outv, sem)
        return carry

    lax.fori_loop(0, (_NCHUNK - 1) // 2, outer, 0)
    drain(out_a, sem_a)
    compute_chunk(jnp.int32(_NCHUNK - 1), out_a)
    store_chunk(jnp.int32(_NCHUNK - 1), out_a, sem_a)
    drain(out_a, sem_a)
    drain(out_b, sem_b)


_gather = functools.partial(
    pl.kernel,
    out_type=jax.ShapeDtypeStruct((_ASP, _L, _B), jnp.float32),
    mesh=plsc.VectorSubcoreMesh(core_axis_name="c", subcore_axis_name="s"),
    compiler_params=pltpu.CompilerParams(needs_layout_passes=False),
    scratch_types=[
        pltpu.VMEM((_VOCAB * _ASP,), jnp.float32),
        pltpu.VMEM((_BPW * _L,), jnp.int32),
        pltpu.VMEM((_ASP, _LCH, _BPW), jnp.float32),
        pltpu.VMEM((_ASP, _LCH, _BPW), jnp.float32),
        pltpu.SemaphoreType.DMA,
        pltpu.SemaphoreType.DMA,
    ],
)(_gather_body)


def kernel(words, emb_table, W, b):
    proj = pl.pallas_call(
        _proj_body,
        out_shape=jax.ShapeDtypeStruct((_VOCAB, _ASP), jnp.float32),
    )(emb_table, W, b.reshape(1, _ASP))
    out_t = _gather(proj.reshape(_VOCAB * _ASP), words.reshape(_B * _L))
    return jnp.transpose(out_t, (2, 1, 0))


# speedup vs baseline: 46.9944x; 1.2117x over previous
import functools

import jax
import jax.numpy as jnp
from jax import lax
from jax.experimental import pallas as pl
from jax.experimental.pallas import tpu as pltpu
from jax.experimental.pallas import tpu_sc as plsc

_VOCAB, _EMB, _ASP = 1000, 64, 9
_B, _L = 4096, 200

_INFO = plsc.get_sparse_core_info()
_NC, _NS = _INFO.num_cores, _INFO.num_subcores
_NW = _NC * _NS
_LANES = 16
_BPW = _B // _NW
_LCH = 8
_NCHUNK = _L // _LCH
_NBG = _BPW // _LANES


def _proj_body(emb_ref, w_ref, b_ref, out_ref):
    out_ref[...] = (
        jnp.dot(emb_ref[...], w_ref[...], preferred_element_type=jnp.float32)
        + b_ref[...]
    )


def _gather_body(proj_hbm, wordsT_hbm, out_hbm,
                 proj_v, idx_v, out_a, out_b, sem_a, sem_b):
    wid = lax.axis_index("s") * _NC + lax.axis_index("c")
    b0 = wid * _BPW

    pltpu.sync_copy(proj_hbm, proj_v)
    pltpu.sync_copy(wordsT_hbm.at[:, pl.ds(b0, _BPW)], idx_v)

    def compute_chunk(lc, outv):
        l0 = lc * _LCH

        @plsc.parallel_loop(0, _NBG, unroll=4)
        def _(bg):
            for l in range(_LCH):
                tok = idx_v[l0 + l, pl.ds(bg * _LANES, _LANES)]
                t9 = tok * _ASP
                for a in range(_ASP):
                    vals = plsc.load_gather(proj_v, [t9 + a])
                    outv[a, l, pl.ds(bg * _LANES, _LANES)] = vals

    def store_chunk(lc, outv, sem):
        pltpu.async_copy(
            outv, out_hbm.at[:, pl.ds(lc * _LCH, _LCH), pl.ds(b0, _BPW)], sem
        )

    def drain(outv, sem):
        pltpu.make_async_copy(
            out_hbm.at[:, pl.ds(0, _LCH), pl.ds(0, _BPW)], outv, sem
        ).wait()

    def outer(p, carry):
        for par in range(2):
            lc = p * 2 + par
            outv = out_a if par == 0 else out_b
            sem = sem_a if par == 0 else sem_b

            @pl.when(p > 0)
            def _():
                drain(outv, sem)

            compute_chunk(lc, outv)
            store_chunk(lc, outv, sem)
        return carry

    lax.fori_loop(0, (_NCHUNK - 1) // 2, outer, 0)
    drain(out_a, sem_a)
    compute_chunk(jnp.int32(_NCHUNK - 1), out_a)
    store_chunk(jnp.int32(_NCHUNK - 1), out_a, sem_a)
    drain(out_a, sem_a)
    drain(out_b, sem_b)


_gather = functools.partial(
    pl.kernel,
    out_type=jax.ShapeDtypeStruct((_ASP, _L, _B), jnp.float32),
    mesh=plsc.VectorSubcoreMesh(core_axis_name="c", subcore_axis_name="s"),
    compiler_params=pltpu.CompilerParams(needs_layout_passes=False),
    scratch_types=[
        pltpu.VMEM((_VOCAB * _ASP,), jnp.float32),
        pltpu.VMEM((_L, _BPW), jnp.int32),
        pltpu.VMEM((_ASP, _LCH, _BPW), jnp.float32),
        pltpu.VMEM((_ASP, _LCH, _BPW), jnp.float32),
        pltpu.SemaphoreType.DMA,
        pltpu.SemaphoreType.DMA,
    ],
)(_gather_body)


def kernel(words, emb_table, W, b):
    proj = pl.pallas_call(
        _proj_body,
        out_shape=jax.ShapeDtypeStruct((_VOCAB, _ASP), jnp.float32),
    )(emb_table, W, b.reshape(1, _ASP))
    out_t = _gather(proj.reshape(_VOCAB * _ASP), jnp.transpose(words))
    return jnp.transpose(out_t, (2, 1, 0))
